# baseline (speedup 1.0000x reference)
import dataclasses
import functools

import jax
import jax.numpy as jnp
from jax import lax
from jax.experimental import pallas as pl
from jax.experimental.pallas import tpu as pltpu
from jax.experimental.pallas import tpu_sc as plsc

N = 10000
E = 320000
F = 128
NT = 32
BPT = 320
NP = NT * BPT
C = 128
NL = 16
IB = 4096
SB = IB // C

_mesh = plsc.VectorSubcoreMesh(core_axis_name="c", subcore_axis_name="s")

_sc_params = pltpu.CompilerParams()
if "needs_layout_passes" in pltpu.CompilerParams.__dataclass_fields__:
    _sc_params = dataclasses.replace(_sc_params, needs_layout_passes=False)
_sc_params = dataclasses.replace(_sc_params, use_tc_tiling_on_sc=False)


def _make_segmin_sc():
    NLB = 2 * NL
    FH = F // 2
    NQ = FH // NL

    @functools.partial(
        pl.kernel,
        mesh=_mesh,
        compiler_params=_sc_params,
        out_type=jax.ShapeDtypeStruct((NP, FH), jnp.int32),
        scratch_types=[
            pltpu.VMEM((BPT + 1, FH), jnp.int32),
            pltpu.VMEM((IB,), jnp.int32),
            pltpu.VMEM((IB,), jnp.int32),
            pltpu.VMEM((2, C, FH), jnp.int32),
            pltpu.VMEM((NT, NL), jnp.int32),
            pltpu.VMEM_SHARED((NP, FH), jnp.int32),
            pltpu.SemaphoreType.DMA,
            pltpu.SemaphoreType.DMA,
        ],
    )
    def k(xsrc_hbm, e0_hbm, e1_hbm, st_hbm, out_hbm,
          acc, e0big, e1big, rows, stm, shared, sem0, sem1):
        sid = lax.axis_index("s")
        wid = sid * 2 + lax.axis_index("c")
        base = wid * BPT

        stripe = NP // 16
        pltpu.sync_copy(xsrc_hbm.at[pl.ds(sid * stripe, stripe)],
                        shared.at[pl.ds(sid * stripe, stripe)])
        infw = plsc.bitcast(jnp.full((NLB,), jnp.inf, jnp.bfloat16),
                            jnp.int32)
        iota16 = lax.iota(jnp.int32, NL)

        @pl.loop(0, BPT + 1)
        def _(r):
            for j in range(NQ):
                acc[r, pl.ds(NL * j, NL)] = infw

        pltpu.sync_copy(st_hbm, stm)
        sv = stm[wid]
        start = sv[0]
        end = sv[1]
        p0 = (start // C) * C
        nch = (end - p0 + (C - 1)) // C
        nsc = (nch + (SB - 1)) // SB
        sems = (sem0, sem1)

        def issue(c, b):
            coff = pl.multiple_of(c * C, C)
            pltpu.make_async_copy(
                shared.at[e0big.at[pl.ds(coff, C)]],
                rows.at[b], sems[b]).start()

        def process(s, c, b):
            coff = pl.multiple_of(c * C, C)
            pltpu.make_async_copy(
                shared.at[e0big.at[pl.ds(coff, C)]],
                rows.at[b], sems[b]).wait()
            chunkpos = p0 + s * IB + c * C

            def dsts(g):
                goff = pl.multiple_of(coff + g * NL, NL)
                dv = e1big[pl.ds(goff, NL)]
                gidx = chunkpos + g * NL + iota16
                ok = (gidx >= start) & (gidx < end)
                return jnp.where(ok, dv - base, BPT)

            d0 = dsts(0)[0]
            avs0 = tuple(
                plsc.bitcast(acc[d0, pl.ds(NL * q, NL)], jnp.bfloat16)
                for q in range(NQ))

            def group(g, carry):
                d_prev = carry[0]
                avs = list(carry[1:])
                d_vec = dsts(g)
                for j in range(NL):
                    d = d_vec[j]
                    sw = d != d_prev
                    for q in range(NQ):
                        sl = pl.ds(NL * q, NL)
                        r = plsc.bitcast(rows[b, g * NL + j, sl],
                                         jnp.bfloat16)
                        avs[q] = jnp.where(sw, r, jnp.minimum(avs[q], r))
                        acc[d, sl] = plsc.bitcast(avs[q], jnp.int32)
                    d_prev = d
                return (d_prev, *avs)

            lax.fori_loop(0, C // NL, group, (d0, *avs0))

        plsc.subcore_barrier()

        @pl.when(nch > 0)
        def _():
            @pl.loop(0, nsc)
            def _(s):
                spos = p0 + s * IB
                pltpu.sync_copy(e0_hbm.at[pl.ds(spos, IB)], e0big)
                pltpu.sync_copy(e1_hbm.at[pl.ds(spos, IB)], e1big)
                ncs = jnp.minimum(SB, nch - s * SB)
                issue(0, 0)

                @pl.loop(0, ncs, step=2)
                def _(c0):
                    for b in range(2):
                        c = c0 + b

                        @pl.when(c < ncs)
                        def _():
                            @pl.when(c + 1 < ncs)
                            def _():
                                issue(c + 1, 1 - b)

                            process(s, c, b)

        pltpu.sync_copy(acc.at[pl.ds(0, BPT)], out_hbm.at[pl.ds(base, BPT)])

    return k


_segmin_sc = _make_segmin_sc()


def _linblock_tc(x, W, b2):
    BN = 1024

    def body(x_ref, w_ref, b_ref, o_ref, ob_ref):
        h = jnp.dot(x_ref[...], w_ref[...],
                    preferred_element_type=jnp.float32,
                    precision=lax.Precision.HIGHEST) + b_ref[...]
        o = jnp.where(h >= 0, h, 0.01 * h)
        o_ref[...] = o
        ob_ref[...] = o.astype(jnp.bfloat16)

    return pl.pallas_call(
        body,
        grid=(pl.cdiv(NP, BN),),
        in_specs=[
            pl.BlockSpec((BN, F), lambda i: (i, 0)),
            pl.BlockSpec((F, F), lambda i: (0, 0)),
            pl.BlockSpec((1, F), lambda i: (0, 0)),
        ],
        out_specs=[pl.BlockSpec((BN, F), lambda i: (i, 0)),
                   pl.BlockSpec((BN, F), lambda i: (i, 0))],
        out_shape=[jax.ShapeDtypeStruct((NP, F), jnp.float32),
                   jax.ShapeDtypeStruct((NP, F), jnp.bfloat16)],
    )(x, W, b2)


def _conv_update_tc(xd, mins, W1, W2, b2):
    BN = 1024

    def body(xd_ref, mn_ref, w1_ref, w2_ref, b_ref, o_ref, ob_ref):
        xdv = xd_ref[...]
        mn = mn_ref[...].astype(jnp.float32)
        mx = jnp.where(mn == jnp.inf, 0.0, xdv - mn)
        h = (jnp.dot(xdv, w1_ref[...], preferred_element_type=jnp.float32,
                     precision=lax.Precision.HIGHEST)
             + jnp.dot(mx, w2_ref[...], preferred_element_type=jnp.float32,
                       precision=lax.Precision.HIGHEST)
             + b_ref[...])
        o = xdv + jnp.where(h >= 0, h, 0.01 * h)
        o_ref[...] = o
        ob_ref[...] = o.astype(jnp.bfloat16)

    return pl.pallas_call(
        body,
        grid=(pl.cdiv(NP, BN),),
        in_specs=[
            pl.BlockSpec((BN, F), lambda i: (i, 0)),
            pl.BlockSpec((BN, F), lambda i: (i, 0)),
            pl.BlockSpec((F, F), lambda i: (0, 0)),
            pl.BlockSpec((F, F), lambda i: (0, 0)),
            pl.BlockSpec((1, F), lambda i: (0, 0)),
        ],
        out_specs=[pl.BlockSpec((BN, F), lambda i: (i, 0)),
                   pl.BlockSpec((BN, F), lambda i: (i, 0))],
        out_shape=[jax.ShapeDtypeStruct((NP, F), jnp.float32),
                   jax.ShapeDtypeStruct((NP, F), jnp.bfloat16)],
    )(xd, mins, W1, W2, b2)


def _prep_edges(el, srow, drow):
    e1 = el[drow].astype(jnp.int32)
    e0 = el[srow].astype(jnp.int32)
    e1s, e0s = lax.sort((e1, e0), num_keys=1)
    bounds = (jnp.arange(1, NT + 1, dtype=jnp.int32)) * BPT
    st = jnp.searchsorted(e1s, bounds, side="left").astype(jnp.int32)
    st33 = jnp.concatenate([jnp.zeros((1,), jnp.int32), st])
    pairs = jnp.stack([st33[:NT], st33[1:]], axis=1)
    starts = jnp.pad(pairs, ((0, 0), (0, NL - 2)))
    e0p = jnp.concatenate([e0s, jnp.zeros((IB,), jnp.int32)])
    e1p = jnp.concatenate([e1s, jnp.full((IB,), NP, jnp.int32)])
    return e0p, e1p, starts


def _pack_bf16(xb):
    return lax.bitcast_convert_type(xb.reshape(NP, F // 2, 2), jnp.int32)


def _unpack_bf16(w):
    return lax.bitcast_convert_type(w, jnp.bfloat16).reshape(NP, F)


def _pad_embed_input(x):
    x = x.astype(jnp.float32)
    x = jnp.pad(x, ((0, NP - x.shape[0]), (0, F - x.shape[1])))
    return x


def _pad_embed_w(W):
    return jnp.pad(W.astype(jnp.float32), ((0, F - W.shape[0]), (0, 0)))


def kernel(vertices, edges, faces, Wv, bv, We, be, Wf, bf,
           Wv2e, bv2e, We2f, be2f, Wm, bm,
           edge_to_vertex, face_to_edge, face_to_face):
    x_v, x_vb = _linblock_tc(_pad_embed_input(vertices), _pad_embed_w(Wv),
                             bv.reshape(1, F))
    x_e, x_eb = _linblock_tc(_pad_embed_input(edges), _pad_embed_w(We),
                             be.reshape(1, F))
    x_f, x_fb = _linblock_tc(_pad_embed_input(faces), _pad_embed_w(Wf),
                             bf.reshape(1, F))

    def do_conv(xsrc_b, xdst, eps, W, b):
        mins = _unpack_bf16(_segmin_sc(_pack_bf16(xsrc_b), *eps))
        return _conv_update_tc(xdst, mins, W[:F], W[F:], b.reshape(1, F))

    eps_v2e = _prep_edges(edge_to_vertex, 1, 0)
    eps_e2f = _prep_edges(face_to_edge, 1, 0)

    x_e, x_eb = do_conv(x_vb, x_e, eps_v2e, Wv2e, bv2e)
    ff_dep, _ = lax.optimization_barrier((face_to_face, x_e))
    eps_ff = _prep_edges(ff_dep, 0, 1)
    x_f, x_fb = do_conv(x_eb, x_f, eps_e2f, We2f, be2f)
    for i in range(Wm.shape[0]):
        x_f, x_fb = do_conv(x_fb, x_f, eps_ff, Wm[i], bm[i])
    return x_f[:N]


# speedup vs baseline: 1.0192x; 1.0192x over previous
import dataclasses
import functools

import jax
import jax.numpy as jnp
from jax import lax
from jax.experimental import pallas as pl
from jax.experimental.pallas import tpu as pltpu
from jax.experimental.pallas import tpu_sc as plsc

N = 10000
E = 320000
F = 128
NT = 32
BPT = 320
NP = NT * BPT
C = 128
NL = 16
IB = 4096
SB = IB // C

_mesh = plsc.VectorSubcoreMesh(core_axis_name="c", subcore_axis_name="s")

_sc_params = pltpu.CompilerParams()
if "needs_layout_passes" in pltpu.CompilerParams.__dataclass_fields__:
    _sc_params = dataclasses.replace(_sc_params, needs_layout_passes=False)
_sc_params = dataclasses.replace(_sc_params, use_tc_tiling_on_sc=False)


def _make_segmin_sc():
    NLB = 2 * NL
    FH = F // 2
    NQ = FH // NL

    @functools.partial(
        pl.kernel,
        mesh=_mesh,
        compiler_params=_sc_params,
        out_type=jax.ShapeDtypeStruct((NP, FH), jnp.int32),
        scratch_types=[
            pltpu.VMEM((BPT + 1, FH), jnp.int32),
            pltpu.VMEM((IB,), jnp.int32),
            pltpu.VMEM((IB,), jnp.int32),
            pltpu.VMEM((2, C, FH), jnp.int32),
            pltpu.VMEM((NT, NL), jnp.int32),
            pltpu.VMEM_SHARED((NP, FH), jnp.int32),
            pltpu.SemaphoreType.DMA,
            pltpu.SemaphoreType.DMA,
        ],
    )
    def k(xsrc_hbm, e0_hbm, e1_hbm, st_hbm, out_hbm,
          acc, e0big, e1big, rows, stm, shared, sem0, sem1):
        sid = lax.axis_index("s")
        wid = sid * 2 + lax.axis_index("c")
        base = wid * BPT

        stripe = NP // 16
        pltpu.sync_copy(xsrc_hbm.at[pl.ds(sid * stripe, stripe)],
                        shared.at[pl.ds(sid * stripe, stripe)])
        infw = plsc.bitcast(jnp.full((NLB,), jnp.inf, jnp.bfloat16),
                            jnp.int32)
        iota16 = lax.iota(jnp.int32, NL)

        @pl.loop(0, BPT + 1)
        def _(r):
            for j in range(NQ):
                acc[r, pl.ds(NL * j, NL)] = infw

        pltpu.sync_copy(st_hbm, stm)
        sv = stm[wid]
        start = sv[0]
        end = sv[1]
        p0 = (start // C) * C
        nch = (end - p0 + (C - 1)) // C
        nsc = (nch + (SB - 1)) // SB
        sems = (sem0, sem1)

        def issue(c, b):
            coff = pl.multiple_of(c * C, C)
            pltpu.make_async_copy(
                shared.at[e0big.at[pl.ds(coff, C)]],
                rows.at[b], sems[b]).start()

        def process(s, c, b):
            coff = pl.multiple_of(c * C, C)
            pltpu.make_async_copy(
                shared.at[e0big.at[pl.ds(coff, C)]],
                rows.at[b], sems[b]).wait()
            chunkpos = p0 + s * IB + c * C

            def dsts(g):
                goff = pl.multiple_of(coff + g * NL, NL)
                dv = e1big[pl.ds(goff, NL)]
                gidx = chunkpos + g * NL + iota16
                ok = (gidx >= start) & (gidx < end)
                return jnp.where(ok, dv - base, BPT)

            d0 = dsts(0)[0]
            avs0 = tuple(
                plsc.bitcast(acc[d0, pl.ds(NL * q, NL)], jnp.bfloat16)
                for q in range(NQ))

            def group(g, carry):
                d_prev = carry[0]
                avs = list(carry[1:])
                d_vec = dsts(g)
                for j in range(NL):
                    d = d_vec[j]
                    sw = d != d_prev
                    for q in range(NQ):
                        sl = pl.ds(NL * q, NL)
                        r = plsc.bitcast(rows[b, g * NL + j, sl],
                                         jnp.bfloat16)
                        avs[q] = jnp.where(sw, r, jnp.minimum(avs[q], r))
                        acc[d, sl] = plsc.bitcast(avs[q], jnp.int32)
                    d_prev = d
                return (d_prev, *avs)

            lax.fori_loop(0, C // NL, group, (d0, *avs0))

        plsc.subcore_barrier()

        @pl.when(nch > 0)
        def _():
            @pl.loop(0, nsc)
            def _(s):
                spos = p0 + s * IB
                pltpu.sync_copy(e0_hbm.at[pl.ds(spos, IB)], e0big)
                pltpu.sync_copy(e1_hbm.at[pl.ds(spos, IB)], e1big)
                ncs = jnp.minimum(SB, nch - s * SB)
                issue(0, 0)

                @pl.loop(0, ncs, step=2)
                def _(c0):
                    for b in range(2):
                        c = c0 + b

                        @pl.when(c < ncs)
                        def _():
                            @pl.when(c + 1 < ncs)
                            def _():
                                issue(c + 1, 1 - b)

                            process(s, c, b)

        pltpu.sync_copy(acc.at[pl.ds(0, BPT)], out_hbm.at[pl.ds(base, BPT)])

    return k


_segmin_sc = _make_segmin_sc()


def _linblock_tc(x, W, b2):
    BN = 1024

    def body(x_ref, w_ref, b_ref, o_ref, ob_ref):
        h = jnp.dot(x_ref[...], w_ref[...],
                    preferred_element_type=jnp.float32,
                    precision=lax.Precision.HIGHEST) + b_ref[...]
        o = jnp.where(h >= 0, h, 0.01 * h)
        o_ref[...] = o
        ob_ref[...] = o.astype(jnp.bfloat16)

    return pl.pallas_call(
        body,
        grid=(pl.cdiv(NP, BN),),
        in_specs=[
            pl.BlockSpec((BN, F), lambda i: (i, 0)),
            pl.BlockSpec((F, F), lambda i: (0, 0)),
            pl.BlockSpec((1, F), lambda i: (0, 0)),
        ],
        out_specs=[pl.BlockSpec((BN, F), lambda i: (i, 0)),
                   pl.BlockSpec((BN, F), lambda i: (i, 0))],
        out_shape=[jax.ShapeDtypeStruct((NP, F), jnp.float32),
                   jax.ShapeDtypeStruct((NP, F), jnp.bfloat16)],
    )(x, W, b2)


def _conv_update_tc(xd, mins, W1, W2, b2):
    BN = 1024

    def body(xd_ref, mn_ref, w1_ref, w2_ref, b_ref, o_ref, ob_ref):
        xdv = xd_ref[...]
        mn = mn_ref[...].astype(jnp.float32)
        mx = jnp.where(mn == jnp.inf, 0.0, xdv - mn)
        h = (jnp.dot(xdv, w1_ref[...], preferred_element_type=jnp.float32,
                     precision=lax.Precision.HIGHEST)
             + jnp.dot(mx, w2_ref[...], preferred_element_type=jnp.float32,
                       precision=lax.Precision.HIGHEST)
             + b_ref[...])
        o = xdv + jnp.where(h >= 0, h, 0.01 * h)
        o_ref[...] = o
        ob_ref[...] = o.astype(jnp.bfloat16)

    return pl.pallas_call(
        body,
        grid=(pl.cdiv(NP, BN),),
        in_specs=[
            pl.BlockSpec((BN, F), lambda i: (i, 0)),
            pl.BlockSpec((BN, F), lambda i: (i, 0)),
            pl.BlockSpec((F, F), lambda i: (0, 0)),
            pl.BlockSpec((F, F), lambda i: (0, 0)),
            pl.BlockSpec((1, F), lambda i: (0, 0)),
        ],
        out_specs=[pl.BlockSpec((BN, F), lambda i: (i, 0)),
                   pl.BlockSpec((BN, F), lambda i: (i, 0))],
        out_shape=[jax.ShapeDtypeStruct((NP, F), jnp.float32),
                   jax.ShapeDtypeStruct((NP, F), jnp.bfloat16)],
    )(xd, mins, W1, W2, b2)


def _prep_edges(el, srow, drow):
    e1 = el[drow].astype(jnp.int32)
    e0 = el[srow].astype(jnp.int32)
    ks = lax.sort((e1 << 14) | e0)
    e1s = ks >> 14
    e0s = ks & 16383
    bounds = (jnp.arange(1, NT + 1, dtype=jnp.int32)) * BPT
    st = jnp.searchsorted(e1s, bounds, side="left").astype(jnp.int32)
    st33 = jnp.concatenate([jnp.zeros((1,), jnp.int32), st])
    pairs = jnp.stack([st33[:NT], st33[1:]], axis=1)
    starts = jnp.pad(pairs, ((0, 0), (0, NL - 2)))
    e0p = jnp.concatenate([e0s, jnp.zeros((IB,), jnp.int32)])
    e1p = jnp.concatenate([e1s, jnp.full((IB,), NP, jnp.int32)])
    return e0p, e1p, starts


def _pack_bf16(xb):
    return lax.bitcast_convert_type(xb.reshape(NP, F // 2, 2), jnp.int32)


def _unpack_bf16(w):
    return lax.bitcast_convert_type(w, jnp.bfloat16).reshape(NP, F)


def _pad_embed_input(x):
    x = x.astype(jnp.float32)
    x = jnp.pad(x, ((0, NP - x.shape[0]), (0, F - x.shape[1])))
    return x


def _pad_embed_w(W):
    return jnp.pad(W.astype(jnp.float32), ((0, F - W.shape[0]), (0, 0)))


def kernel(vertices, edges, faces, Wv, bv, We, be, Wf, bf,
           Wv2e, bv2e, We2f, be2f, Wm, bm,
           edge_to_vertex, face_to_edge, face_to_face):
    x_v, x_vb = _linblock_tc(_pad_embed_input(vertices), _pad_embed_w(Wv),
                             bv.reshape(1, F))
    x_e, x_eb = _linblock_tc(_pad_embed_input(edges), _pad_embed_w(We),
                             be.reshape(1, F))
    x_f, x_fb = _linblock_tc(_pad_embed_input(faces), _pad_embed_w(Wf),
                             bf.reshape(1, F))

    def do_conv(xsrc_b, xdst, eps, W, b):
        mins = _unpack_bf16(_segmin_sc(_pack_bf16(xsrc_b), *eps))
        return _conv_update_tc(xdst, mins, W[:F], W[F:], b.reshape(1, F))

    eps_v2e = _prep_edges(edge_to_vertex, 1, 0)
    eps_e2f = _prep_edges(face_to_edge, 1, 0)

    x_e, x_eb = do_conv(x_vb, x_e, eps_v2e, Wv2e, bv2e)
    ff_dep, _ = lax.optimization_barrier((face_to_face, x_e))
    eps_ff = _prep_edges(ff_dep, 0, 1)
    x_f, x_fb = do_conv(x_eb, x_f, eps_e2f, We2f, be2f)
    for i in range(Wm.shape[0]):
        x_f, x_fb = do_conv(x_fb, x_f, eps_ff, Wm[i], bm[i])
    return x_f[:N]
